# Initial kernel scaffold; baseline (speedup 1.0000x reference)
import functools

import jax
import jax.numpy as jnp
from jax import lax
from jax.experimental import pallas as pl
from jax.experimental.pallas import tpu as pltpu
from jax.experimental.pallas import tpu_sc as plsc

N0, N1, N2 = 10000, 4000, 1000
E0, E1 = 64000, 16000
D_IN, D_HID, D_OUT = 256, 512, 256

NC, NS = 2, 16


def _make_agg(n_src, n_dst, d, c_chunks, k):
  rz = -(-n_dst // NS)
  zch = min(k, rz)
  nzc = -(-rz // zch)
  mesh = plsc.VectorSubcoreMesh(core_axis_name="c", subcore_axis_name="s")

  @functools.partial(
      pl.kernel,
      out_type=[
          jax.ShapeDtypeStruct((NC, n_dst, d), jnp.float32),
          jax.ShapeDtypeStruct((NC, n_dst, 16), jnp.float32),
      ],
      mesh=mesh,
      scratch_types=[
          pltpu.VMEM_SHARED((n_dst, d), jnp.float32),
          pltpu.VMEM_SHARED((n_dst, 16), jnp.float32),
          pltpu.VMEM((c_chunks, k), jnp.int32),
          pltpu.VMEM((c_chunks, k), jnp.int32),
          pltpu.VMEM((k, d), jnp.float32),
          pltpu.VMEM((k, 16), jnp.float32),
          pltpu.VMEM((rz, 16), jnp.float32),
          pltpu.SemaphoreType.DMA,
      ],
  )
  def agg(table, srcr, dstr, out_acc, out_cnt,
          acc_sh, cnt_sh, src_vm, dst_vm, gbuf, ones_vm, zc_vm, gsem):
    cid = lax.axis_index("c")
    sid = lax.axis_index("s")

    def zrow(i, carry):
      for t in range(d // 16):
        gbuf[i, pl.ds(t * 16, 16)] = jnp.zeros((16,), jnp.float32)
      return carry
    lax.fori_loop(0, k, zrow, 0)

    def zrow16(i, carry):
      zc_vm[i, :] = jnp.zeros((16,), jnp.float32)
      return carry
    lax.fori_loop(0, rz, zrow16, 0)

    def orow16(i, carry):
      ones_vm[i, :] = jnp.ones((16,), jnp.float32)
      return carry
    lax.fori_loop(0, k, orow16, 0)

    start = jnp.minimum(sid * rz, n_dst - rz)
    for q in range(nzc):
      st = jnp.minimum(start + q * zch, n_dst - zch)
      pltpu.sync_copy(gbuf.at[pl.ds(0, zch)], acc_sh.at[pl.ds(st, zch)])
    pltpu.sync_copy(zc_vm, cnt_sh.at[pl.ds(start, rz)])

    pltpu.sync_copy(srcr.at[cid, sid], src_vm)
    pltpu.sync_copy(dstr.at[cid, sid], dst_vm)

    plsc.subcore_barrier()

    def step(j, carry):
      pltpu.async_copy(table.at[src_vm.at[j]], gbuf, gsem).wait()
      pltpu.sync_copy(gbuf, acc_sh.at[dst_vm.at[j]], add=True)
      pltpu.sync_copy(ones_vm, cnt_sh.at[dst_vm.at[j]], add=True)
      return carry
    lax.fori_loop(0, c_chunks, step, 0)

    plsc.subcore_barrier()

    for q in range(nzc):
      st = jnp.minimum(start + q * zch, n_dst - zch)
      pltpu.sync_copy(acc_sh.at[pl.ds(st, zch)], out_acc.at[cid, pl.ds(st, zch)])
    pltpu.sync_copy(cnt_sh.at[pl.ds(start, rz)], out_cnt.at[cid, pl.ds(start, rz)])

  return agg


_agg1 = _make_agg(N0, N1, D_IN, 16, 125)
_agg2 = _make_agg(N1, N2, D_HID, 5, 100)


def _dense_body(acc_ref, cnt_ref, w_ref, b_ref, out_ref):
  s = acc_ref[0] + acc_ref[1]
  c = cnt_ref[0, :, 0:1] + cnt_ref[1, :, 0:1]
  mean = s / jnp.maximum(c, 1.0)
  y = jnp.dot(mean, w_ref[...], preferred_element_type=jnp.float32)
  out_ref[...] = jnp.maximum(y + b_ref[...], 0.0)


def _dense(acc, cnt, w, b):
  n = acc.shape[1]
  dout = w.shape[1]
  return pl.pallas_call(
      _dense_body,
      out_shape=jax.ShapeDtypeStruct((n, dout), jnp.float32),
  )(acc, cnt, w, b)


def kernel(x, src0, dst0, src1, dst1, W1, b1, W2, b2):
  src0r = src0.astype(jnp.int32).reshape(NC, NS, 16, 125)
  dst0r = dst0.astype(jnp.int32).reshape(NC, NS, 16, 125)
  src1r = src1.astype(jnp.int32).reshape(NC, NS, 5, 100)
  dst1r = dst1.astype(jnp.int32).reshape(NC, NS, 5, 100)

  acc1, cnt1 = _agg1(x, src0r, dst0r)
  h = _dense(acc1, cnt1, W1, b1.reshape(1, D_HID))
  acc2, cnt2 = _agg2(h, src1r, dst1r)
  return _dense(acc2, cnt2, W2, b2.reshape(1, D_OUT))


# speedup vs baseline: 3.1237x; 3.1237x over previous
import functools

import jax
import jax.numpy as jnp
from jax import lax
from jax.experimental import pallas as pl
from jax.experimental.pallas import tpu as pltpu
from jax.experimental.pallas import tpu_sc as plsc

N0, N1, N2 = 10000, 4000, 1000
E0, E1 = 64000, 16000
D_IN, D_HID, D_OUT = 256, 512, 256

NC, NS = 2, 16
K = 128


def _make_agg(n_src, n_dst, d, c_chunks):
  n_pad = n_dst + 8
  rz = -(-n_dst // NS)
  rz = -(-rz // 8) * 8
  zch = min(K, rz)
  nzc = rz // zch
  mesh = plsc.VectorSubcoreMesh(core_axis_name="c", subcore_axis_name="s")

  @functools.partial(
      pl.kernel,
      out_type=[
          jax.ShapeDtypeStruct((NC, n_dst, d), jnp.float32),
          jax.ShapeDtypeStruct((NC, n_dst, 16), jnp.float32),
      ],
      mesh=mesh,
      compiler_params=pltpu.CompilerParams(use_tc_tiling_on_sc=False),
      scratch_types=[
          pltpu.VMEM_SHARED((n_pad, d), jnp.float32),
          pltpu.VMEM_SHARED((n_pad, 16), jnp.float32),
          pltpu.VMEM((c_chunks, K), jnp.int32),
          pltpu.VMEM((c_chunks, K), jnp.int32),
          pltpu.VMEM((K, d), jnp.float32),
          pltpu.VMEM((K, 16), jnp.float32),
          pltpu.VMEM((rz, 16), jnp.float32),
          pltpu.SemaphoreType.DMA,
      ],
  )
  def agg(table, srcr, dstr, out_acc, out_cnt,
          acc_sh, cnt_sh, src_vm, dst_vm, gbuf, ones_vm, zc_vm, gsem):
    cid = lax.axis_index("c")
    sid = lax.axis_index("s")

    def zrow(i, carry):
      for t in range(d // 16):
        gbuf[i, pl.ds(t * 16, 16)] = jnp.zeros((16,), jnp.float32)
      return carry
    lax.fori_loop(0, K, zrow, 0)

    def zrow16(i, carry):
      zc_vm[i, :] = jnp.zeros((16,), jnp.float32)
      return carry
    lax.fori_loop(0, rz, zrow16, 0)

    def orow16(i, carry):
      ones_vm[i, :] = jnp.ones((16,), jnp.float32)
      return carry
    lax.fori_loop(0, K, orow16, 0)

    start = pl.multiple_of(jnp.minimum(sid * rz, n_dst - rz), 8)
    for q in range(nzc):
      st = pl.multiple_of(jnp.minimum(start + q * zch, n_dst - zch), 8)
      pltpu.sync_copy(gbuf.at[pl.ds(0, zch)], acc_sh.at[pl.ds(st, zch)])
    pltpu.sync_copy(zc_vm, cnt_sh.at[pl.ds(start, rz)])

    pltpu.sync_copy(srcr.at[cid, sid], src_vm)
    pltpu.sync_copy(dstr.at[cid, sid], dst_vm)

    plsc.subcore_barrier()

    def step(j, carry):
      pltpu.async_copy(table.at[src_vm.at[j]], gbuf, gsem).wait()
      pltpu.sync_copy(gbuf, acc_sh.at[dst_vm.at[j]], add=True)
      pltpu.sync_copy(ones_vm, cnt_sh.at[dst_vm.at[j]], add=True)
      return carry
    lax.fori_loop(0, c_chunks, step, 0)

    plsc.subcore_barrier()

    for q in range(nzc):
      st = pl.multiple_of(jnp.minimum(start + q * zch, n_dst - zch), 8)
      pltpu.sync_copy(acc_sh.at[pl.ds(st, zch)], out_acc.at[cid, pl.ds(st, zch)])
    pltpu.sync_copy(cnt_sh.at[pl.ds(start, rz)], out_cnt.at[cid, pl.ds(start, rz)])

  return agg


_agg1 = _make_agg(N0, N1, D_IN, 16)
_agg2 = _make_agg(N1, N2, D_HID, 4)


def _pad_edges(src, dst, c_chunks, dummy_dst):
  e_pad = NC * NS * c_chunks * K
  pad = e_pad - src.shape[0]
  src_p = jnp.concatenate([src.astype(jnp.int32),
                           jnp.zeros((pad,), jnp.int32)])
  dst_p = jnp.concatenate([dst.astype(jnp.int32),
                           jnp.full((pad,), dummy_dst, jnp.int32)])
  return (src_p.reshape(NC, NS, c_chunks, K),
          dst_p.reshape(NC, NS, c_chunks, K))


def _dense_body(acc_ref, cnt_ref, w_ref, b_ref, out_ref):
  s = acc_ref[0] + acc_ref[1]
  c = cnt_ref[0, :, 0:1] + cnt_ref[1, :, 0:1]
  mean = s / jnp.maximum(c, 1.0)
  y = jnp.dot(mean, w_ref[...], preferred_element_type=jnp.float32)
  out_ref[...] = jnp.maximum(y + b_ref[...], 0.0)


def _dense(acc, cnt, w, b):
  n = acc.shape[1]
  dout = w.shape[1]
  return pl.pallas_call(
      _dense_body,
      out_shape=jax.ShapeDtypeStruct((n, dout), jnp.float32),
  )(acc, cnt, w, b)


def kernel(x, src0, dst0, src1, dst1, W1, b1, W2, b2):
  src0r, dst0r = _pad_edges(src0, dst0, 16, N1)
  src1r, dst1r = _pad_edges(src1, dst1, 4, N2)

  acc1, cnt1 = _agg1(x, src0r, dst0r)
  h = _dense(acc1, cnt1, W1, b1.reshape(1, D_HID))
  acc2, cnt2 = _agg2(h, src1r, dst1r)
  return _dense(acc2, cnt2, W2, b2.reshape(1, D_OUT))
